# Initial kernel scaffold; baseline (speedup 1.0000x reference)
import jax
import jax.numpy as jnp
from jax.experimental import pallas as pl


def kernel(x, edge_index, W1_l, b1_l, W1_r, W2_l, b2_l, W2_r):
    raise NotImplementedError("write your pallas kernel here")


# speedup vs baseline: 7.5323x; 7.5323x over previous
import functools

import jax
import jax.numpy as jnp
from jax import lax
from jax.experimental import pallas as pl
from jax.experimental.pallas import tpu as pltpu
from jax.experimental.pallas import tpu_sc as plsc

N = 10000
E = 320000
D = 128

NC = 2
NS = 16
NW = NC * NS
EPW = E // NW
CH = 80
NCH = EPW // CH
RPS = 632
RLAST = N - 15 * RPS


def _make_sc_segment_sum(with_counts):
  out_type = [jax.ShapeDtypeStruct((NC, N, D), jnp.float32)]
  scratch = [
      pltpu.VMEM((NCH, CH), jnp.int32),
      pltpu.VMEM((NCH, CH), jnp.int32),
      pltpu.VMEM((CH, D), jnp.float32),
      pltpu.VMEM_SHARED((N, D), jnp.float32),
      pltpu.SemaphoreType.DMA,
  ]
  if with_counts:
    out_type.append(jax.ShapeDtypeStruct((NC, 1, N), jnp.float32))
    scratch += [
        pltpu.VMEM((CH,), jnp.float32),
        pltpu.VMEM_SHARED((N,), jnp.float32),
    ]

  def body(x_hbm, src_hbm, dst_hbm, z_hbm, zc_hbm, *rest):
    if with_counts:
      (out_hbm, cnt_hbm, src_v, dst_v, rows_v, acc_sh, sem,
       ones_v, cnt_sh) = rest
    else:
      out_hbm, src_v, dst_v, rows_v, acc_sh, sem = rest

    c = lax.axis_index("c")
    s = lax.axis_index("s")
    wid = s * NC + c
    base = pl.multiple_of(s * RPS, 8)

    @pl.when(s < NS - 1)
    def _():
      pltpu.sync_copy(z_hbm, acc_sh.at[pl.ds(base, RPS)])

    @pl.when(s == NS - 1)
    def _():
      pltpu.sync_copy(z_hbm.at[pl.ds(0, RLAST)], acc_sh.at[pl.ds(base, RLAST)])

    if with_counts:
      one16 = jnp.ones((16,), jnp.float32)
      def ones_body(i, _):
        ones_v[pl.ds(i * 16, 16)] = one16
        return 0
      lax.fori_loop(0, CH // 16, ones_body, 0)

      @pl.when(s == 0)
      def _():
        pltpu.sync_copy(zc_hbm.at[0], cnt_sh)

    pltpu.sync_copy(src_hbm.at[wid], src_v)
    pltpu.sync_copy(dst_hbm.at[wid], dst_v)

    plsc.subcore_barrier()

    def chunk_body(j, _):
      pltpu.async_copy(x_hbm.at[src_v.at[j]], rows_v, sem).wait()
      pltpu.sync_copy(rows_v, acc_sh.at[dst_v.at[j]], add=True)
      if with_counts:
        pltpu.sync_copy(ones_v, cnt_sh.at[dst_v.at[j]], add=True)
      return 0
    lax.fori_loop(0, NCH, chunk_body, 0)

    plsc.subcore_barrier()

    @pl.when(s < NS - 1)
    def _():
      pltpu.sync_copy(acc_sh.at[pl.ds(base, RPS)], out_hbm.at[c, pl.ds(base, RPS)])

    @pl.when(s == NS - 1)
    def _():
      pltpu.sync_copy(acc_sh.at[pl.ds(base, RLAST)],
                      out_hbm.at[c, pl.ds(base, RLAST)])

    if with_counts:
      @pl.when(s == 0)
      def _():
        pltpu.sync_copy(cnt_sh, cnt_hbm.at[c, 0])

  mesh = plsc.VectorSubcoreMesh(core_axis_name="c", subcore_axis_name="s")
  return pl.kernel(body, out_type=out_type, mesh=mesh, scratch_types=scratch)


_sc_sum_counts = _make_sc_segment_sum(True)
_sc_sum = _make_sc_segment_sum(False)


BN = 400


def _dense_body(p_ref, inv_ref, x_ref, wl_ref, wr_ref, b_ref, o_ref):
  agg = (p_ref[0] + p_ref[1]) * inv_ref[...]
  o_ref[...] = jnp.maximum(
      jnp.dot(agg, wl_ref[...], preferred_element_type=jnp.float32)
      + jnp.dot(x_ref[...], wr_ref[...], preferred_element_type=jnp.float32)
      + b_ref[...], 0.0)


def _dense(partials, invb, x, W_l, W_r, b_l):
  grid = (N // BN,)
  return pl.pallas_call(
      _dense_body,
      grid=grid,
      in_specs=[
          pl.BlockSpec((NC, BN, D), lambda i: (0, i, 0)),
          pl.BlockSpec((BN, D), lambda i: (i, 0)),
          pl.BlockSpec((BN, D), lambda i: (i, 0)),
          pl.BlockSpec((D, D), lambda i: (0, 0)),
          pl.BlockSpec((D, D), lambda i: (0, 0)),
          pl.BlockSpec((1, D), lambda i: (0, 0)),
      ],
      out_specs=pl.BlockSpec((BN, D), lambda i: (i, 0)),
      out_shape=jax.ShapeDtypeStruct((N, D), jnp.float32),
      compiler_params=pltpu.CompilerParams(
          dimension_semantics=("parallel",)),
  )(partials, invb, x, W_l, W_r, b_l)


def kernel(x, edge_index, W1_l, b1_l, W1_r, W2_l, b2_l, W2_r):
  src = edge_index[0].reshape(NW, NCH, CH)
  dst = edge_index[1].reshape(NW, NCH, CH)

  z = jnp.zeros((RPS, D), jnp.float32)
  zc = jnp.zeros((1, N), jnp.float32)

  sums1, cnts = _sc_sum_counts(x, src, dst, z, zc)
  cnt = jnp.maximum(cnts[0, 0] + cnts[1, 0], 1.0)
  invb = jnp.broadcast_to((1.0 / cnt)[:, None], (N, D))

  h = _dense(sums1, invb, x, W1_l, W1_r, b1_l.reshape(1, D))
  (sums2,) = _sc_sum(h, src, dst, z, zc)
  out = _dense(sums2, invb, h, W2_l, W2_r, b2_l.reshape(1, D))
  return out


# speedup vs baseline: 11.0409x; 1.4658x over previous
import functools

import jax
import jax.numpy as jnp
from jax import lax
from jax.experimental import pallas as pl
from jax.experimental.pallas import tpu as pltpu
from jax.experimental.pallas import tpu_sc as plsc

N = 10000
E = 320000
D = 128

NC = 2
NS = 16
NW = NC * NS
EPW = E // NW
CH = 80
NCH = EPW // CH
NB = 5
NCHB = NCH // NB
RPS = 632
RLAST = N - 15 * RPS


def _make_sc_segment_sum(with_counts):
  out_type = [jax.ShapeDtypeStruct((NC, N, D), jnp.float32)]
  scratch = [
      pltpu.VMEM((NCHB, CH), jnp.int32),
      pltpu.VMEM((NCHB, CH), jnp.int32),
      pltpu.VMEM((CH, D), jnp.float32),
      pltpu.VMEM((CH, D), jnp.float32),
      pltpu.VMEM_SHARED((N, D), jnp.float32),
      pltpu.SemaphoreType.DMA,
      pltpu.SemaphoreType.DMA,
  ]
  if with_counts:
    out_type.append(jax.ShapeDtypeStruct((NC, 1, N), jnp.float32))
    scratch += [
        pltpu.VMEM((CH,), jnp.float32),
        pltpu.VMEM_SHARED((N,), jnp.float32),
    ]

  def body(x_hbm, src_hbm, dst_hbm, z_hbm, zc_hbm, *rest):
    if with_counts:
      (out_hbm, cnt_hbm, src_v, dst_v, rows0, rows1, acc_sh, sem0, sem1,
       ones_v, cnt_sh) = rest
    else:
      out_hbm, src_v, dst_v, rows0, rows1, acc_sh, sem0, sem1 = rest

    c = lax.axis_index("c")
    s = lax.axis_index("s")
    wid = s * NC + c
    base = pl.multiple_of(s * RPS, 8)

    @pl.when(s < NS - 1)
    def _():
      pltpu.sync_copy(z_hbm, acc_sh.at[pl.ds(base, RPS)])

    @pl.when(s == NS - 1)
    def _():
      pltpu.sync_copy(z_hbm.at[pl.ds(0, RLAST)], acc_sh.at[pl.ds(base, RLAST)])

    if with_counts:
      one16 = jnp.ones((16,), jnp.float32)
      def ones_body(i, _):
        ones_v[pl.ds(i * 16, 16)] = one16
        return 0
      lax.fori_loop(0, CH // 16, ones_body, 0)

      @pl.when(s == 0)
      def _():
        pltpu.sync_copy(zc_hbm.at[0], cnt_sh)

    plsc.subcore_barrier()

    def gather(j, buf, sem):
      pltpu.async_copy(x_hbm.at[src_v.at[j]], buf, sem)

    def wait_gather(j, buf, sem):
      pltpu.make_async_copy(x_hbm.at[src_v.at[j]], buf, sem).wait()

    def scatter(j, buf):
      pltpu.sync_copy(buf, acc_sh.at[dst_v.at[j]], add=True)
      if with_counts:
        pltpu.sync_copy(ones_v, cnt_sh.at[dst_v.at[j]], add=True)

    for b in range(NB):
      pltpu.sync_copy(src_hbm.at[wid, b], src_v)
      pltpu.sync_copy(dst_hbm.at[wid, b], dst_v)

      gather(0, rows0, sem0)

      def chunk_body(jj, _):
        j0 = jj * 2
        gather(j0 + 1, rows1, sem1)
        wait_gather(j0, rows0, sem0)
        scatter(j0, rows0)
        gather(j0 + 2, rows0, sem0)
        wait_gather(j0 + 1, rows1, sem1)
        scatter(j0 + 1, rows1)
        return 0
      lax.fori_loop(0, (NCHB - 1) // 2, chunk_body, 0)

      wait_gather(NCHB - 1, rows0, sem0)
      scatter(NCHB - 1, rows0)

    plsc.subcore_barrier()

    @pl.when(s < NS - 1)
    def _():
      pltpu.sync_copy(acc_sh.at[pl.ds(base, RPS)], out_hbm.at[c, pl.ds(base, RPS)])

    @pl.when(s == NS - 1)
    def _():
      pltpu.sync_copy(acc_sh.at[pl.ds(base, RLAST)],
                      out_hbm.at[c, pl.ds(base, RLAST)])

    if with_counts:
      @pl.when(s == 0)
      def _():
        pltpu.sync_copy(cnt_sh, cnt_hbm.at[c, 0])

  mesh = plsc.VectorSubcoreMesh(core_axis_name="c", subcore_axis_name="s")
  return pl.kernel(body, out_type=out_type, mesh=mesh, scratch_types=scratch)


_sc_sum_counts = _make_sc_segment_sum(True)
_sc_sum = _make_sc_segment_sum(False)


BN = 400


def _dense_body(p_ref, inv_ref, x_ref, wl_ref, wr_ref, b_ref, o_ref):
  agg = (p_ref[0] + p_ref[1]) * inv_ref[...]
  o_ref[...] = jnp.maximum(
      jnp.dot(agg, wl_ref[...], preferred_element_type=jnp.float32)
      + jnp.dot(x_ref[...], wr_ref[...], preferred_element_type=jnp.float32)
      + b_ref[...], 0.0)


def _dense(partials, invb, x, W_l, W_r, b_l):
  grid = (N // BN,)
  return pl.pallas_call(
      _dense_body,
      grid=grid,
      in_specs=[
          pl.BlockSpec((NC, BN, D), lambda i: (0, i, 0)),
          pl.BlockSpec((BN, D), lambda i: (i, 0)),
          pl.BlockSpec((BN, D), lambda i: (i, 0)),
          pl.BlockSpec((D, D), lambda i: (0, 0)),
          pl.BlockSpec((D, D), lambda i: (0, 0)),
          pl.BlockSpec((1, D), lambda i: (0, 0)),
      ],
      out_specs=pl.BlockSpec((BN, D), lambda i: (i, 0)),
      out_shape=jax.ShapeDtypeStruct((N, D), jnp.float32),
      compiler_params=pltpu.CompilerParams(
          dimension_semantics=("parallel",)),
  )(partials, invb, x, W_l, W_r, b_l)


def kernel(x, edge_index, W1_l, b1_l, W1_r, W2_l, b2_l, W2_r):
  src = edge_index[0].reshape(NW, NB, NCHB, CH)
  dst = edge_index[1].reshape(NW, NB, NCHB, CH)

  z = jnp.zeros((RPS, D), jnp.float32)
  zc = jnp.zeros((1, N), jnp.float32)

  sums1, cnts = _sc_sum_counts(x, src, dst, z, zc)
  cnt = jnp.maximum(cnts[0, 0] + cnts[1, 0], 1.0)
  invb = jnp.broadcast_to((1.0 / cnt)[:, None], (N, D))

  h = _dense(sums1, invb, x, W1_l, W1_r, b1_l.reshape(1, D))
  (sums2,) = _sc_sum(h, src, dst, z, zc)
  out = _dense(sums2, invb, h, W2_l, W2_r, b2_l.reshape(1, D))
  return out


# speedup vs baseline: 11.1642x; 1.0112x over previous
import functools

import jax
import jax.numpy as jnp
from jax import lax
from jax.experimental import pallas as pl
from jax.experimental.pallas import tpu as pltpu
from jax.experimental.pallas import tpu_sc as plsc

N = 10000
E = 320000
D = 128

NC = 2
NS = 16
NW = NC * NS
EPW = E // NW
CH = 80
NCH = EPW // CH
NB = 5
NCHB = NCH // NB
RPS = 632
RLAST = N - 15 * RPS


def _make_sc_segment_sum(with_counts):
  out_type = [jax.ShapeDtypeStruct((NC, N, D), jnp.float32)]
  scratch = [
      pltpu.VMEM((NCHB, CH), jnp.int32),
      pltpu.VMEM((NCHB, CH), jnp.int32),
      pltpu.VMEM((CH, D), jnp.float32),
      pltpu.VMEM((CH, D), jnp.float32),
      pltpu.VMEM_SHARED((N, D), jnp.float32),
      pltpu.SemaphoreType.DMA,
      pltpu.SemaphoreType.DMA,
  ]
  if with_counts:
    out_type.append(jax.ShapeDtypeStruct((NC, 1, N), jnp.float32))
    scratch += [
        pltpu.VMEM((CH,), jnp.float32),
        pltpu.VMEM_SHARED((N,), jnp.float32),
    ]

  def body(x_hbm, src_hbm, dst_hbm, z_hbm, zc_hbm, *rest):
    if with_counts:
      (out_hbm, cnt_hbm, src_v, dst_v, rows0, rows1, acc_sh, sem0, sem1,
       ones_v, cnt_sh) = rest
    else:
      out_hbm, src_v, dst_v, rows0, rows1, acc_sh, sem0, sem1 = rest

    c = lax.axis_index("c")
    s = lax.axis_index("s")
    wid = s * NC + c
    base = pl.multiple_of(s * RPS, 8)

    @pl.when(s < NS - 1)
    def _():
      pltpu.sync_copy(z_hbm, acc_sh.at[pl.ds(base, RPS)])

    @pl.when(s == NS - 1)
    def _():
      pltpu.sync_copy(z_hbm.at[pl.ds(0, RLAST)], acc_sh.at[pl.ds(base, RLAST)])

    if with_counts:
      one16 = jnp.ones((16,), jnp.float32)
      def ones_body(i, _):
        ones_v[pl.ds(i * 16, 16)] = one16
        return 0
      lax.fori_loop(0, CH // 16, ones_body, 0)

      @pl.when(s == 0)
      def _():
        pltpu.sync_copy(zc_hbm.at[0], cnt_sh)

    plsc.subcore_barrier()

    def gather(j, buf, sem):
      pltpu.async_copy(x_hbm.at[src_v.at[j]], buf, sem)

    def wait_gather(j, buf, sem):
      pltpu.make_async_copy(x_hbm.at[src_v.at[j]], buf, sem).wait()

    def scatter(j, buf):
      pltpu.sync_copy(buf, acc_sh.at[dst_v.at[j]], add=True)
      if with_counts:
        pltpu.sync_copy(ones_v, cnt_sh.at[dst_v.at[j]], add=True)

    for b in range(NB):
      pltpu.sync_copy(src_hbm.at[wid, b], src_v)
      pltpu.sync_copy(dst_hbm.at[wid, b], dst_v)

      gather(0, rows0, sem0)

      def chunk_body(jj, _):
        j0 = jj * 2
        gather(j0 + 1, rows1, sem1)
        wait_gather(j0, rows0, sem0)
        scatter(j0, rows0)
        gather(j0 + 2, rows0, sem0)
        wait_gather(j0 + 1, rows1, sem1)
        scatter(j0 + 1, rows1)
        return 0
      lax.fori_loop(0, (NCHB - 1) // 2, chunk_body, 0)

      wait_gather(NCHB - 1, rows0, sem0)
      scatter(NCHB - 1, rows0)

    plsc.subcore_barrier()

    @pl.when(s < NS - 1)
    def _():
      pltpu.sync_copy(acc_sh.at[pl.ds(base, RPS)], out_hbm.at[c, pl.ds(base, RPS)])

    @pl.when(s == NS - 1)
    def _():
      pltpu.sync_copy(acc_sh.at[pl.ds(base, RLAST)],
                      out_hbm.at[c, pl.ds(base, RLAST)])

    if with_counts:
      @pl.when(s == 0)
      def _():
        pltpu.sync_copy(cnt_sh, cnt_hbm.at[c, 0])

  mesh = plsc.VectorSubcoreMesh(core_axis_name="c", subcore_axis_name="s")
  return pl.kernel(body, out_type=out_type, mesh=mesh, scratch_types=scratch)


_sc_sum_counts = _make_sc_segment_sum(True)
_sc_sum = _make_sc_segment_sum(False)


BN = 400


def _dense_body(p_ref, inv_ref, x_ref, wl_ref, wr_ref, b_ref, o_ref):
  agg = (p_ref[0] + p_ref[1]) * inv_ref[...]
  o_ref[...] = jnp.maximum(
      jnp.dot(agg, wl_ref[...], preferred_element_type=jnp.float32)
      + jnp.dot(x_ref[...], wr_ref[...], preferred_element_type=jnp.float32)
      + b_ref[...], 0.0)


def _dense(partials, invb, x, W_l, W_r, b_l):
  grid = (N // BN,)
  return pl.pallas_call(
      _dense_body,
      grid=grid,
      in_specs=[
          pl.BlockSpec((NC, BN, D), lambda i: (0, i, 0)),
          pl.BlockSpec((BN, 1), lambda i: (i, 0)),
          pl.BlockSpec((BN, D), lambda i: (i, 0)),
          pl.BlockSpec((D, D), lambda i: (0, 0)),
          pl.BlockSpec((D, D), lambda i: (0, 0)),
          pl.BlockSpec((1, D), lambda i: (0, 0)),
      ],
      out_specs=pl.BlockSpec((BN, D), lambda i: (i, 0)),
      out_shape=jax.ShapeDtypeStruct((N, D), jnp.float32),
      compiler_params=pltpu.CompilerParams(
          dimension_semantics=("parallel",)),
  )(partials, invb, x, W_l, W_r, b_l)


def kernel(x, edge_index, W1_l, b1_l, W1_r, W2_l, b2_l, W2_r):
  src = edge_index[0].reshape(NW, NB, NCHB, CH)
  dst = edge_index[1].reshape(NW, NB, NCHB, CH)

  z = jnp.zeros((RPS, D), jnp.float32)
  zc = jnp.zeros((1, N), jnp.float32)

  sums1, cnts = _sc_sum_counts(x, src, dst, z, zc)
  inv = (1.0 / jnp.maximum(cnts[0, 0] + cnts[1, 0], 1.0))[:, None]

  h = _dense(sums1, inv, x, W1_l, W1_r, b1_l.reshape(1, D))
  (sums2,) = _sc_sum(h, src, dst, z, zc)
  out = _dense(sums2, inv, h, W2_l, W2_r, b2_l.reshape(1, D))
  return out
